# baseline (speedup 1.0000x reference)
import jax
import jax.numpy as jnp
from jax import lax
from jax.experimental import pallas as pl
from jax.experimental.pallas import tpu as pltpu
from jax.experimental.pallas import tpu_sc as plsc

_N0, _E0 = 10000, 320000
_N1, _E1 = 2500, 40000
_D = 128
_DA = 144
_NC, _NS = 2, 16
_NW = _NC * _NS
_CH0 = 80
_DG = 8
_CH1 = 128
_CC = 80
_N0P = 10240
_N0A = 10152
_N1P = 2560
_E0P = 323584
_E1P = 40960
_DW = 16
_BM0 = 400
_BM1 = 256

_mesh = plsc.VectorSubcoreMesh(core_axis_name="c", subcore_axis_name="s",
                               num_cores=_NC, num_subcores=_NS)
_sc_params = pltpu.CompilerParams(use_tc_tiling_on_sc=False)


def _seg_body(nch_w, rpt, nclch_w):
    npair = nch_w // 2
    tail = nch_w % 2

    def body(tab_hbm, src_hbm, dst_hbm, cl_hbm, ones_hbm, z128_hbm, z16_hbm,
             agg_out, deg_out, degc_out,
             sidx, didx, clidx, bufa, bufb, ones_v,
             acc_sh, deg_sh, degc_sh, sem):
        cid = lax.axis_index("c")
        sid = lax.axis_index("s")
        wid = cid * _NS + sid
        pltpu.sync_copy(z128_hbm.at[pl.ds(0, rpt)],
                        acc_sh.at[pl.ds(sid * rpt, rpt)])
        pltpu.sync_copy(z16_hbm.at[pl.ds(0, rpt)],
                        deg_sh.at[pl.ds(sid * rpt, rpt)])
        pltpu.sync_copy(z16_hbm.at[pl.ds(0, rpt)],
                        degc_sh.at[pl.ds(sid * rpt, rpt)])
        pltpu.sync_copy(cl_hbm.at[pl.ds(wid * nclch_w, nclch_w)], clidx)
        pltpu.sync_copy(ones_hbm, ones_v)
        pltpu.sync_copy(src_hbm.at[pl.ds(wid * nch_w, nch_w)], sidx)
        pltpu.sync_copy(dst_hbm.at[pl.ds(wid * nch_w, nch_w)], didx)
        plsc.subcore_barrier()

        def scat(buf, c):
            pltpu.sync_copy(buf, acc_sh.at[didx.at[c]], add=True)
            pltpu.sync_copy(ones_v, deg_sh.at[didx.at[c]], add=True)

        pltpu.async_copy(tab_hbm.at[sidx.at[0]], bufa, sem).wait()

        def pair(p, carry):
            c0 = 2 * p
            d1 = pltpu.async_copy(tab_hbm.at[sidx.at[c0 + 1]], bufb, sem)
            scat(bufa, c0)
            d1.wait()
            cnxt = jnp.minimum(c0 + 2, nch_w - 1)
            d2 = pltpu.async_copy(tab_hbm.at[sidx.at[cnxt]], bufa, sem)
            scat(bufb, c0 + 1)
            d2.wait()
            return carry

        lax.fori_loop(0, npair, pair, 0)
        if tail:
            scat(bufa, nch_w - 1)

        def cstep(q, carry):
            pltpu.sync_copy(ones_v.at[pl.ds(0, _CC)],
                            degc_sh.at[clidx.at[q]], add=True)
            return carry
        lax.fori_loop(0, nclch_w, cstep, 0)

        plsc.subcore_barrier()
        pltpu.sync_copy(acc_sh.at[pl.ds(sid * rpt, rpt)],
                        agg_out.at[cid].at[pl.ds(sid * rpt, rpt)])
        pltpu.sync_copy(deg_sh.at[pl.ds(sid * rpt, rpt)],
                        deg_out.at[cid].at[pl.ds(sid * rpt, rpt)])
        pltpu.sync_copy(degc_sh.at[pl.ds(sid * rpt, rpt)],
                        degc_out.at[cid].at[pl.ds(sid * rpt, rpt)])
    return body


def _g0_body(nch_w, rpt):
    npair = nch_w // 2
    tail = nch_w % 2

    def body(tab_hbm, src_hbm, dst_hbm, z128_hbm, z8_hbm, ones_hbm,
             agg_out, deg_out, sidx, didx, bufa, bufb, ones_v,
             acc_sh, deg_sh, sem):
        cid = lax.axis_index("c")
        sid = lax.axis_index("s")
        wid = cid * _NS + sid
        pltpu.sync_copy(z128_hbm.at[pl.ds(0, rpt)],
                        acc_sh.at[pl.ds(sid * rpt, rpt)])
        pltpu.sync_copy(z8_hbm.at[pl.ds(0, rpt)],
                        deg_sh.at[pl.ds(sid * rpt, rpt)])
        pltpu.sync_copy(ones_hbm, ones_v)
        pltpu.sync_copy(src_hbm.at[pl.ds(wid * nch_w, nch_w)], sidx)
        pltpu.sync_copy(dst_hbm.at[pl.ds(wid * nch_w, nch_w)], didx)
        plsc.subcore_barrier()

        def scat(buf, c):
            pltpu.sync_copy(buf, acc_sh.at[didx.at[c]], add=True)
            pltpu.sync_copy(ones_v, deg_sh.at[didx.at[c]], add=True)

        pltpu.async_copy(tab_hbm.at[sidx.at[0]], bufa, sem).wait()

        def pair(p, carry):
            c0 = 2 * p
            d1 = pltpu.async_copy(tab_hbm.at[sidx.at[c0 + 1]], bufb, sem)
            scat(bufa, c0)
            d1.wait()
            cnxt = jnp.minimum(c0 + 2, nch_w - 1)
            d2 = pltpu.async_copy(tab_hbm.at[sidx.at[cnxt]], bufa, sem)
            scat(bufb, c0 + 1)
            d2.wait()
            return carry

        lax.fori_loop(0, npair, pair, 0)
        if tail:
            scat(bufa, nch_w - 1)
        plsc.subcore_barrier()
        pltpu.sync_copy(acc_sh.at[pl.ds(sid * rpt, rpt)],
                        agg_out.at[cid].at[pl.ds(sid * rpt, rpt)])
        pltpu.sync_copy(deg_sh.at[pl.ds(sid * rpt, rpt)],
                        deg_out.at[cid].at[pl.ds(sid * rpt, rpt)])
    return body


def _inter_body(nch_w, rpt):
    npair = nch_w // 2

    def body(h1_hbm, h0_hbm, cl_hbm, z128_hbm, g1_out, csum_out,
             cidx, ba1, ba0, bb1, bb0, csum_sh, sem1, sem0):
        cid = lax.axis_index("c")
        sid = lax.axis_index("s")
        wid = cid * _NS + sid
        pltpu.sync_copy(z128_hbm.at[pl.ds(0, rpt)],
                        csum_sh.at[pl.ds(sid * rpt, rpt)])
        pltpu.sync_copy(cl_hbm.at[pl.ds(wid * nch_w, nch_w)], cidx)
        plsc.subcore_barrier()

        def issue(c, b1, b0):
            d1 = pltpu.async_copy(h1_hbm.at[cidx.at[c]], b1, sem1)
            d0 = pltpu.async_copy(h0_hbm.at[pl.ds((wid * nch_w + c) * _CC,
                                                  _CC)], b0, sem0)
            return d1, d0

        def drain(c, b1, b0):
            pltpu.sync_copy(b1, g1_out.at[pl.ds((wid * nch_w + c) * _CC,
                                                _CC)])
            pltpu.sync_copy(b0, csum_sh.at[cidx.at[c]], add=True)

        da, db = issue(0, ba1, ba0)
        da.wait()
        db.wait()

        def pair(p, carry):
            c0 = 2 * p
            d1, d0 = issue(c0 + 1, bb1, bb0)
            drain(c0, ba1, ba0)
            d1.wait()
            d0.wait()
            cnxt = jnp.minimum(c0 + 2, nch_w - 1)
            d1, d0 = issue(cnxt, ba1, ba0)
            drain(c0 + 1, bb1, bb0)
            d1.wait()
            d0.wait()
            return carry

        lax.fori_loop(0, npair, pair, 0)
        plsc.subcore_barrier()
        pltpu.sync_copy(csum_sh.at[pl.ds(sid * rpt, rpt)],
                        csum_out.at[cid].at[pl.ds(sid * rpt, rpt)])
    return body


def _h_body(x_ref, p_ref, ws_ref, wn_ref, o_ref):
    p = p_ref[0] + p_ref[1]
    agg = p[:, :_D] * (1.0 / jnp.maximum(p[:, _D:_D + 1], 1.0))
    o_ref[...] = jnp.maximum(
        jnp.dot(x_ref[...], ws_ref[...], preferred_element_type=jnp.float32)
        + jnp.dot(agg, wn_ref[...], preferred_element_type=jnp.float32), 0.0)


def _h0_body(x_ref, p_ref, d_ref, ws_ref, wn_ref, o_ref):
    deg = d_ref[0] + d_ref[1]
    agg = (p_ref[0] + p_ref[1]) * (1.0 / jnp.maximum(deg[:, :1], 1.0))
    o_ref[...] = jnp.maximum(
        jnp.dot(x_ref[...], ws_ref[...], preferred_element_type=jnp.float32)
        + jnp.dot(agg, wn_ref[...], preferred_element_type=jnp.float32), 0.0)


def _comb0_body(x_ref, h_ref, g_ref, wsi_ref, wni_ref, o_ref):
    h = h_ref[...]
    nz = jnp.maximum(
        jnp.dot(h, wsi_ref[...], preferred_element_type=jnp.float32)
        + jnp.dot(g_ref[...], wni_ref[...], preferred_element_type=jnp.float32),
        0.0)
    o_ref[...] = x_ref[...] + 0.5 * h + 0.5 * nz


def _comb1_body(x_ref, h_ref, c_ref, dc_ref, wsi_ref, wni_ref, o_ref):
    dc = dc_ref[0] + dc_ref[1]
    aggc = (c_ref[0] + c_ref[1]) * (1.0 / jnp.maximum(dc[:, :1], 1.0))
    h = h_ref[...]
    nz = jnp.maximum(
        jnp.dot(h, wsi_ref[...], preferred_element_type=jnp.float32)
        + jnp.dot(aggc, wni_ref[...], preferred_element_type=jnp.float32),
        0.0)
    o_ref[...] = x_ref[...] + 0.5 * h + 0.5 * nz


def _row_spec(bm, d):
    return pl.BlockSpec((bm, d), lambda i: (i, 0))


def _part_spec(bm, d):
    return pl.BlockSpec((2, bm, d), lambda i: (0, i, 0))


def _w_spec():
    return pl.BlockSpec((_D, _D), lambda i: (0, 0))


def _tc_h(xp, parts, ws, wn, n_rows, bm):
    return pl.pallas_call(
        _h_body,
        grid=(n_rows // bm,),
        in_specs=[_row_spec(bm, _D), _part_spec(bm, _DA),
                  _w_spec(), _w_spec()],
        out_specs=_row_spec(bm, _D),
        out_shape=jax.ShapeDtypeStruct((n_rows, _D), jnp.float32),
    )(xp, parts, ws, wn)


def kernel(x0, x1, edge_index0, edge_index1, inter_edge_index,
           W_self0, W_neigh0, W_self1, W_neigh1, W_self_i, W_neigh_i):
    f32 = jnp.float32
    x1p = jnp.pad(x1, ((0, _N1P - _N1), (0, 0)))
    pad1 = _N1 + jnp.arange(_E1P - _E1, dtype=jnp.int32) % (_N1P - _N1)
    padc = _N1 + jnp.arange(_N0P - _N0, dtype=jnp.int32) % (_N1P - _N1)
    src0 = edge_index0[0].reshape(_E0 // _CH0, _CH0)
    dst0 = edge_index0[1].reshape(_E0 // _CH0, _CH0)
    src1 = jnp.pad(edge_index1[0], (0, _E1P - _E1)).reshape(_E1P // _CH1, _CH1)
    dst1 = jnp.concatenate([edge_index1[1], pad1]).reshape(_E1P // _CH1, _CH1)
    cluster = inter_edge_index[1, :_N0] - _N0
    clp = jnp.concatenate([cluster, padc]).reshape(_N0P // _CC, _CC)
    ones16 = jnp.ones((_CH1, _DW), f32)
    ones8 = jnp.ones((_CH0, _DG), f32)
    rpt1 = _N1P // _NS
    rpt0 = _N0 // _NS
    z16 = jnp.zeros((rpt1, _DW), f32)
    z8 = jnp.zeros((rpt0, _DG), f32)
    z128 = jnp.zeros((rpt0, _D), f32)

    nch1 = _E1P // _CH1 // _NW
    nchc = _N0P // _CC // _NW
    nch0 = _E0 // _CH0 // _NW

    agg1, deg1, degc = pl.kernel(
        _seg_body(nch1, rpt1, nchc),
        out_type=(jax.ShapeDtypeStruct((_NC, _N1P, _D), f32),
                  jax.ShapeDtypeStruct((_NC, _N1P, _DW), f32),
                  jax.ShapeDtypeStruct((_NC, _N1P, _DW), f32)),
        mesh=_mesh,
        scratch_types=[
            pltpu.VMEM((nch1, _CH1), jnp.int32),
            pltpu.VMEM((nch1, _CH1), jnp.int32),
            pltpu.VMEM((nchc, _CC), jnp.int32),
            pltpu.VMEM((_CH1, _D), f32),
            pltpu.VMEM((_CH1, _D), f32),
            pltpu.VMEM((_CH1, _DW), f32),
            pltpu.VMEM_SHARED((_N1P, _D), f32),
            pltpu.VMEM_SHARED((_N1P, _DW), f32),
            pltpu.VMEM_SHARED((_N1P, _DW), f32),
            pltpu.SemaphoreType.DMA,
        ],
        name="sc_seg_g1",
        compiler_params=_sc_params,
    )(x1, src1, dst1, clp, ones16, z128, z16)

    agg0, deg0 = pl.kernel(
        _g0_body(nch0, rpt0),
        out_type=(jax.ShapeDtypeStruct((_NC, _N0, _D), f32),
                  jax.ShapeDtypeStruct((_NC, _N0, _DG), f32)),
        mesh=_mesh,
        scratch_types=[
            pltpu.VMEM((nch0, _CH0), jnp.int32),
            pltpu.VMEM((nch0, _CH0), jnp.int32),
            pltpu.VMEM((_CH0, _D), f32),
            pltpu.VMEM((_CH0, _D), f32),
            pltpu.VMEM((_CH0, _DG), f32),
            pltpu.VMEM_SHARED((_N0, _D), f32),
            pltpu.VMEM_SHARED((_N0, _DG), f32),
            pltpu.SemaphoreType.DMA,
        ],
        name="sc_seg_g0",
        compiler_params=_sc_params,
    )(x0, src0, dst0, z128, z8, ones8)

    h1p = pl.pallas_call(
        _h0_body,
        grid=(_N1P // _BM1,),
        in_specs=[_row_spec(_BM1, _D), _part_spec(_BM1, _D),
                  _part_spec(_BM1, _DW), _w_spec(), _w_spec()],
        out_specs=_row_spec(_BM1, _D),
        out_shape=jax.ShapeDtypeStruct((_N1P, _D), f32),
    )(x1p, agg1, deg1, W_self1, W_neigh1)
    h0 = pl.pallas_call(
        _h0_body,
        grid=(_N0 // _BM0,),
        in_specs=[_row_spec(_BM0, _D), _part_spec(_BM0, _D),
                  _part_spec(_BM0, _DG), _w_spec(), _w_spec()],
        out_specs=_row_spec(_BM0, _D),
        out_shape=jax.ShapeDtypeStruct((_N0, _D), f32),
    )(x0, agg0, deg0, W_self0, W_neigh0)
    h0p = jnp.pad(h0, ((0, _N0P - _N0), (0, 0)))

    g1, csum = pl.kernel(
        _inter_body(nchc, rpt1),
        out_type=(jax.ShapeDtypeStruct((_N0P, _D), f32),
                  jax.ShapeDtypeStruct((_NC, _N1P, _D), f32)),
        mesh=_mesh,
        scratch_types=[
            pltpu.VMEM((nchc, _CC), jnp.int32),
            pltpu.VMEM((_CC, _D), f32),
            pltpu.VMEM((_CC, _D), f32),
            pltpu.VMEM((_CC, _D), f32),
            pltpu.VMEM((_CC, _D), f32),
            pltpu.VMEM_SHARED((_N1P, _D), f32),
            pltpu.SemaphoreType.DMA,
            pltpu.SemaphoreType.DMA,
        ],
        name="sc_inter",
        compiler_params=_sc_params,
    )(h1p, h0p, clp, z128)

    out0 = pl.pallas_call(
        _comb0_body,
        grid=(_N0 // _BM0,),
        in_specs=[_row_spec(_BM0, _D), _row_spec(_BM0, _D),
                  _row_spec(_BM0, _D), _w_spec(), _w_spec()],
        out_specs=_row_spec(_BM0, _D),
        out_shape=jax.ShapeDtypeStruct((_N0, _D), f32),
    )(x0, h0, g1[:_N0], W_self_i, W_neigh_i)

    out1p = pl.pallas_call(
        _comb1_body,
        grid=(_N1P // _BM1,),
        in_specs=[_row_spec(_BM1, _D), _row_spec(_BM1, _D),
                  _part_spec(_BM1, _D), _part_spec(_BM1, _DW),
                  _w_spec(), _w_spec()],
        out_specs=_row_spec(_BM1, _D),
        out_shape=jax.ShapeDtypeStruct((_N1P, _D), f32),
    )(x1p, h1p, csum, degc, W_self_i, W_neigh_i)

    return (out0, out1p[:_N1])


# speedup vs baseline: 1.0841x; 1.0841x over previous
import jax
import jax.numpy as jnp
from jax import lax
from jax.experimental import pallas as pl
from jax.experimental.pallas import tpu as pltpu
from jax.experimental.pallas import tpu_sc as plsc

_N0, _E0 = 10000, 320000
_N1, _E1 = 2500, 40000
_D = 128
_DA = 144
_NC, _NS = 2, 16
_NW = _NC * _NS
_CH0 = 80
_DG = 8
_CH1 = 128
_CC = 80
_N0P = 10240
_N0A = 10152
_N1P = 2560
_E0P = 323584
_E1P = 40960
_DW = 16
_BM0 = 400
_BM1 = 256

_mesh = plsc.VectorSubcoreMesh(core_axis_name="c", subcore_axis_name="s",
                               num_cores=_NC, num_subcores=_NS)
_sc_params = pltpu.CompilerParams(use_tc_tiling_on_sc=False)


def _seg_body(nch_w, rpt, nclch_w):
    npair = nch_w // 2
    tail = nch_w % 2

    def body(tab_hbm, src_hbm, dst_hbm, cl_hbm, ones_hbm, z128_hbm, z16_hbm,
             agg_out, deg_out, degc_out,
             sidx, didx, clidx, bufa, bufb, ones_v,
             acc_sh, deg_sh, degc_sh, sem):
        cid = lax.axis_index("c")
        sid = lax.axis_index("s")
        wid = cid * _NS + sid
        pltpu.sync_copy(z128_hbm.at[pl.ds(0, rpt)],
                        acc_sh.at[pl.ds(sid * rpt, rpt)])
        pltpu.sync_copy(z16_hbm.at[pl.ds(0, rpt)],
                        deg_sh.at[pl.ds(sid * rpt, rpt)])
        pltpu.sync_copy(z16_hbm.at[pl.ds(0, rpt)],
                        degc_sh.at[pl.ds(sid * rpt, rpt)])
        pltpu.sync_copy(cl_hbm.at[pl.ds(wid * nclch_w, nclch_w)], clidx)
        pltpu.sync_copy(ones_hbm, ones_v)
        pltpu.sync_copy(src_hbm.at[pl.ds(wid * nch_w, nch_w)], sidx)
        pltpu.sync_copy(dst_hbm.at[pl.ds(wid * nch_w, nch_w)], didx)
        plsc.subcore_barrier()

        def scat(buf, c):
            pltpu.sync_copy(buf, acc_sh.at[didx.at[c]], add=True)
            pltpu.sync_copy(ones_v, deg_sh.at[didx.at[c]], add=True)

        pltpu.async_copy(tab_hbm.at[sidx.at[0]], bufa, sem).wait()

        def pair(p, carry):
            c0 = 2 * p
            d1 = pltpu.async_copy(tab_hbm.at[sidx.at[c0 + 1]], bufb, sem)
            scat(bufa, c0)
            d1.wait()
            cnxt = jnp.minimum(c0 + 2, nch_w - 1)
            d2 = pltpu.async_copy(tab_hbm.at[sidx.at[cnxt]], bufa, sem)
            scat(bufb, c0 + 1)
            d2.wait()
            return carry

        lax.fori_loop(0, npair, pair, 0)
        if tail:
            scat(bufa, nch_w - 1)

        def cstep(q, carry):
            pltpu.sync_copy(ones_v.at[pl.ds(0, _CC)],
                            degc_sh.at[clidx.at[q]], add=True)
            return carry
        lax.fori_loop(0, nclch_w, cstep, 0)

        plsc.subcore_barrier()
        pltpu.sync_copy(acc_sh.at[pl.ds(sid * rpt, rpt)],
                        agg_out.at[cid].at[pl.ds(sid * rpt, rpt)])
        pltpu.sync_copy(deg_sh.at[pl.ds(sid * rpt, rpt)],
                        deg_out.at[cid].at[pl.ds(sid * rpt, rpt)])
        pltpu.sync_copy(degc_sh.at[pl.ds(sid * rpt, rpt)],
                        degc_out.at[cid].at[pl.ds(sid * rpt, rpt)])
    return body


def _g0_body(nch_w, rpt):
    npair = nch_w // 2
    tail = nch_w % 2

    def body(tab_hbm, src_hbm, dst_hbm, z128_hbm, z8_hbm, ones_hbm,
             agg_out, deg_out, sidx, didx, bufa, bufb, ones_v,
             acc_sh, deg_sh, sem):
        cid = lax.axis_index("c")
        sid = lax.axis_index("s")
        wid = cid * _NS + sid
        pltpu.sync_copy(z128_hbm.at[pl.ds(0, rpt)],
                        acc_sh.at[pl.ds(sid * rpt, rpt)])
        pltpu.sync_copy(z8_hbm.at[pl.ds(0, rpt)],
                        deg_sh.at[pl.ds(sid * rpt, rpt)])
        pltpu.sync_copy(ones_hbm, ones_v)
        pltpu.sync_copy(src_hbm.at[pl.ds(wid * nch_w, nch_w)], sidx)
        pltpu.sync_copy(dst_hbm.at[pl.ds(wid * nch_w, nch_w)], didx)
        plsc.subcore_barrier()

        def scat(buf, c):
            pltpu.sync_copy(buf, acc_sh.at[didx.at[c]], add=True)
            pltpu.sync_copy(ones_v, deg_sh.at[didx.at[c]], add=True)

        pltpu.async_copy(tab_hbm.at[sidx.at[0]], bufa, sem).wait()

        def pair(p, carry):
            c0 = 2 * p
            d1 = pltpu.async_copy(tab_hbm.at[sidx.at[c0 + 1]], bufb, sem)
            scat(bufa, c0)
            d1.wait()
            cnxt = jnp.minimum(c0 + 2, nch_w - 1)
            d2 = pltpu.async_copy(tab_hbm.at[sidx.at[cnxt]], bufa, sem)
            scat(bufb, c0 + 1)
            d2.wait()
            return carry

        lax.fori_loop(0, npair, pair, 0)
        if tail:
            scat(bufa, nch_w - 1)
        plsc.subcore_barrier()
        pltpu.sync_copy(acc_sh.at[pl.ds(sid * rpt, rpt)],
                        agg_out.at[cid].at[pl.ds(sid * rpt, rpt)])
        pltpu.sync_copy(deg_sh.at[pl.ds(sid * rpt, rpt)],
                        deg_out.at[cid].at[pl.ds(sid * rpt, rpt)])
    return body


def _inter_body(nch_w, rpt):
    npair = nch_w // 2

    def body(h1_hbm, h0_hbm, cl_hbm, z128_hbm, g1_out, csum_out,
             cidx, ba1, ba0, bb1, bb0, csum_sh, sem1, sem0):
        cid = lax.axis_index("c")
        sid = lax.axis_index("s")
        wid = cid * _NS + sid
        pltpu.sync_copy(z128_hbm.at[pl.ds(0, rpt)],
                        csum_sh.at[pl.ds(sid * rpt, rpt)])
        pltpu.sync_copy(cl_hbm.at[pl.ds(wid * nch_w, nch_w)], cidx)
        plsc.subcore_barrier()

        def issue(c, b1, b0):
            d1 = pltpu.async_copy(h1_hbm.at[cidx.at[c]], b1, sem1)
            d0 = pltpu.async_copy(h0_hbm.at[pl.ds((wid * nch_w + c) * _CC,
                                                  _CC)], b0, sem0)
            return d1, d0

        def drain(c, b1, b0):
            pltpu.sync_copy(b1, g1_out.at[pl.ds((wid * nch_w + c) * _CC,
                                                _CC)])
            pltpu.sync_copy(b0, csum_sh.at[cidx.at[c]], add=True)

        da, db = issue(0, ba1, ba0)
        da.wait()
        db.wait()

        def pair(p, carry):
            c0 = 2 * p
            d1, d0 = issue(c0 + 1, bb1, bb0)
            drain(c0, ba1, ba0)
            d1.wait()
            d0.wait()
            cnxt = jnp.minimum(c0 + 2, nch_w - 1)
            d1, d0 = issue(cnxt, ba1, ba0)
            drain(c0 + 1, bb1, bb0)
            d1.wait()
            d0.wait()
            return carry

        lax.fori_loop(0, npair, pair, 0)
        plsc.subcore_barrier()
        pltpu.sync_copy(csum_sh.at[pl.ds(sid * rpt, rpt)],
                        csum_out.at[cid].at[pl.ds(sid * rpt, rpt)])
    return body


def _h_body(x_ref, p_ref, ws_ref, wn_ref, o_ref):
    p = p_ref[0] + p_ref[1]
    agg = p[:, :_D] * (1.0 / jnp.maximum(p[:, _D:_D + 1], 1.0))
    o_ref[...] = jnp.maximum(
        jnp.dot(x_ref[...], ws_ref[...], preferred_element_type=jnp.float32)
        + jnp.dot(agg, wn_ref[...], preferred_element_type=jnp.float32), 0.0)


def _h0_body(x_ref, p_ref, d_ref, ws_ref, wn_ref, o_ref):
    deg = d_ref[0] + d_ref[1]
    psum = p_ref[0].astype(jnp.float32) + p_ref[1].astype(jnp.float32)
    agg = psum * (1.0 / jnp.maximum(deg[:, :1], 1.0))
    o_ref[...] = jnp.maximum(
        jnp.dot(x_ref[...], ws_ref[...], preferred_element_type=jnp.float32)
        + jnp.dot(agg, wn_ref[...], preferred_element_type=jnp.float32), 0.0)


def _comb0_body(x_ref, h_ref, g_ref, wsi_ref, wni_ref, o_ref):
    h = h_ref[...]
    nz = jnp.maximum(
        jnp.dot(h, wsi_ref[...], preferred_element_type=jnp.float32)
        + jnp.dot(g_ref[...], wni_ref[...], preferred_element_type=jnp.float32),
        0.0)
    o_ref[...] = x_ref[...] + 0.5 * h + 0.5 * nz


def _comb1_body(x_ref, h_ref, c_ref, dc_ref, wsi_ref, wni_ref, o_ref):
    dc = dc_ref[0] + dc_ref[1]
    aggc = (c_ref[0] + c_ref[1]) * (1.0 / jnp.maximum(dc[:, :1], 1.0))
    h = h_ref[...]
    nz = jnp.maximum(
        jnp.dot(h, wsi_ref[...], preferred_element_type=jnp.float32)
        + jnp.dot(aggc, wni_ref[...], preferred_element_type=jnp.float32),
        0.0)
    o_ref[...] = x_ref[...] + 0.5 * h + 0.5 * nz


def _row_spec(bm, d):
    return pl.BlockSpec((bm, d), lambda i: (i, 0))


def _part_spec(bm, d):
    return pl.BlockSpec((2, bm, d), lambda i: (0, i, 0))


def _w_spec():
    return pl.BlockSpec((_D, _D), lambda i: (0, 0))


def _tc_h(xp, parts, ws, wn, n_rows, bm):
    return pl.pallas_call(
        _h_body,
        grid=(n_rows // bm,),
        in_specs=[_row_spec(bm, _D), _part_spec(bm, _DA),
                  _w_spec(), _w_spec()],
        out_specs=_row_spec(bm, _D),
        out_shape=jax.ShapeDtypeStruct((n_rows, _D), jnp.float32),
    )(xp, parts, ws, wn)


def kernel(x0, x1, edge_index0, edge_index1, inter_edge_index,
           W_self0, W_neigh0, W_self1, W_neigh1, W_self_i, W_neigh_i):
    f32 = jnp.float32
    x1p = jnp.pad(x1, ((0, _N1P - _N1), (0, 0)))
    pad1 = _N1 + jnp.arange(_E1P - _E1, dtype=jnp.int32) % (_N1P - _N1)
    padc = _N1 + jnp.arange(_N0P - _N0, dtype=jnp.int32) % (_N1P - _N1)
    src0 = edge_index0[0].reshape(_E0 // _CH0, _CH0)
    dst0 = edge_index0[1].reshape(_E0 // _CH0, _CH0)
    src1 = jnp.pad(edge_index1[0], (0, _E1P - _E1)).reshape(_E1P // _CH1, _CH1)
    dst1 = jnp.concatenate([edge_index1[1], pad1]).reshape(_E1P // _CH1, _CH1)
    cluster = inter_edge_index[1, :_N0] - _N0
    clp = jnp.concatenate([cluster, padc]).reshape(_N0P // _CC, _CC)
    ones16 = jnp.ones((_CH1, _DW), f32)
    ones8 = jnp.ones((_CH0, _DG), f32)
    rpt1 = _N1P // _NS
    rpt0 = _N0 // _NS
    z16 = jnp.zeros((rpt1, _DW), f32)
    z8 = jnp.zeros((rpt0, _DG), f32)
    z128 = jnp.zeros((rpt0, _D), f32)

    nch1 = _E1P // _CH1 // _NW
    nchc = _N0P // _CC // _NW
    nch0 = _E0 // _CH0 // _NW

    agg1, deg1, degc = pl.kernel(
        _seg_body(nch1, rpt1, nchc),
        out_type=(jax.ShapeDtypeStruct((_NC, _N1P, _D), f32),
                  jax.ShapeDtypeStruct((_NC, _N1P, _DW), f32),
                  jax.ShapeDtypeStruct((_NC, _N1P, _DW), f32)),
        mesh=_mesh,
        scratch_types=[
            pltpu.VMEM((nch1, _CH1), jnp.int32),
            pltpu.VMEM((nch1, _CH1), jnp.int32),
            pltpu.VMEM((nchc, _CC), jnp.int32),
            pltpu.VMEM((_CH1, _D), f32),
            pltpu.VMEM((_CH1, _D), f32),
            pltpu.VMEM((_CH1, _DW), f32),
            pltpu.VMEM_SHARED((_N1P, _D), f32),
            pltpu.VMEM_SHARED((_N1P, _DW), f32),
            pltpu.VMEM_SHARED((_N1P, _DW), f32),
            pltpu.SemaphoreType.DMA,
        ],
        name="sc_seg_g1",
        compiler_params=_sc_params,
    )(x1, src1, dst1, clp, ones16, z128, z16)

    bf16 = jnp.bfloat16
    x0h = x0.astype(bf16)
    zb = jnp.zeros((rpt0, _D), bf16)
    agg0, deg0 = pl.kernel(
        _g0_body(nch0, rpt0),
        out_type=(jax.ShapeDtypeStruct((_NC, _N0, _D), bf16),
                  jax.ShapeDtypeStruct((_NC, _N0, _DG), f32)),
        mesh=_mesh,
        scratch_types=[
            pltpu.VMEM((nch0, _CH0), jnp.int32),
            pltpu.VMEM((nch0, _CH0), jnp.int32),
            pltpu.VMEM((_CH0, _D), bf16),
            pltpu.VMEM((_CH0, _D), bf16),
            pltpu.VMEM((_CH0, _DG), f32),
            pltpu.VMEM_SHARED((_N0, _D), bf16),
            pltpu.VMEM_SHARED((_N0, _DG), f32),
            pltpu.SemaphoreType.DMA,
        ],
        name="sc_seg_g0",
        compiler_params=_sc_params,
    )(x0h, src0, dst0, zb, z8, ones8)

    h1p = pl.pallas_call(
        _h0_body,
        grid=(_N1P // _BM1,),
        in_specs=[_row_spec(_BM1, _D), _part_spec(_BM1, _D),
                  _part_spec(_BM1, _DW), _w_spec(), _w_spec()],
        out_specs=_row_spec(_BM1, _D),
        out_shape=jax.ShapeDtypeStruct((_N1P, _D), f32),
    )(x1p, agg1, deg1, W_self1, W_neigh1)
    h0 = pl.pallas_call(
        _h0_body,
        grid=(_N0 // _BM0,),
        in_specs=[_row_spec(_BM0, _D), _part_spec(_BM0, _D),
                  _part_spec(_BM0, _DG), _w_spec(), _w_spec()],
        out_specs=_row_spec(_BM0, _D),
        out_shape=jax.ShapeDtypeStruct((_N0, _D), f32),
    )(x0, agg0, deg0, W_self0, W_neigh0)
    h0p = jnp.pad(h0, ((0, _N0P - _N0), (0, 0)))

    g1, csum = pl.kernel(
        _inter_body(nchc, rpt1),
        out_type=(jax.ShapeDtypeStruct((_N0P, _D), f32),
                  jax.ShapeDtypeStruct((_NC, _N1P, _D), f32)),
        mesh=_mesh,
        scratch_types=[
            pltpu.VMEM((nchc, _CC), jnp.int32),
            pltpu.VMEM((_CC, _D), f32),
            pltpu.VMEM((_CC, _D), f32),
            pltpu.VMEM((_CC, _D), f32),
            pltpu.VMEM((_CC, _D), f32),
            pltpu.VMEM_SHARED((_N1P, _D), f32),
            pltpu.SemaphoreType.DMA,
            pltpu.SemaphoreType.DMA,
        ],
        name="sc_inter",
        compiler_params=_sc_params,
    )(h1p, h0p, clp, z128)

    out0 = pl.pallas_call(
        _comb0_body,
        grid=(_N0 // _BM0,),
        in_specs=[_row_spec(_BM0, _D), _row_spec(_BM0, _D),
                  _row_spec(_BM0, _D), _w_spec(), _w_spec()],
        out_specs=_row_spec(_BM0, _D),
        out_shape=jax.ShapeDtypeStruct((_N0, _D), f32),
    )(x0, h0, g1[:_N0], W_self_i, W_neigh_i)

    out1p = pl.pallas_call(
        _comb1_body,
        grid=(_N1P // _BM1,),
        in_specs=[_row_spec(_BM1, _D), _row_spec(_BM1, _D),
                  _part_spec(_BM1, _D), _part_spec(_BM1, _DW),
                  _w_spec(), _w_spec()],
        out_specs=_row_spec(_BM1, _D),
        out_shape=jax.ShapeDtypeStruct((_N1P, _D), f32),
    )(x1p, h1p, csum, degc, W_self_i, W_neigh_i)

    return (out0, out1p[:_N1])


# speedup vs baseline: 1.1140x; 1.0276x over previous
import jax
import jax.numpy as jnp
from jax import lax
from jax.experimental import pallas as pl
from jax.experimental.pallas import tpu as pltpu
from jax.experimental.pallas import tpu_sc as plsc

_N0, _E0 = 10000, 320000
_N1, _E1 = 2500, 40000
_D = 128
_DA = 144
_NC, _NS = 2, 16
_NW = _NC * _NS
_CH0 = 80
_DG = 8
_CH1 = 128
_CC = 80
_N0P = 10240
_N0A = 10152
_N1P = 2560
_E0P = 323584
_E1P = 40960
_DW = 16
_BM0 = 400
_BM1 = 256

_mesh = plsc.VectorSubcoreMesh(core_axis_name="c", subcore_axis_name="s",
                               num_cores=_NC, num_subcores=_NS)
_sc_params = pltpu.CompilerParams(use_tc_tiling_on_sc=False)


def _seg_body(nch_w, rpt, nclch_w):
    npair = nch_w // 2
    tail = nch_w % 2

    def body(tab_hbm, src_hbm, dst_hbm, cl_hbm, ones_hbm, z128_hbm, z16_hbm,
             agg_out, deg_out, degc_out,
             sidx, didx, clidx, bufa, bufb, ones_v,
             acc_sh, deg_sh, degc_sh, sem):
        cid = lax.axis_index("c")
        sid = lax.axis_index("s")
        wid = cid * _NS + sid
        pltpu.sync_copy(z128_hbm.at[pl.ds(0, rpt)],
                        acc_sh.at[pl.ds(sid * rpt, rpt)])
        pltpu.sync_copy(z16_hbm.at[pl.ds(0, rpt)],
                        deg_sh.at[pl.ds(sid * rpt, rpt)])
        pltpu.sync_copy(z16_hbm.at[pl.ds(0, rpt)],
                        degc_sh.at[pl.ds(sid * rpt, rpt)])
        pltpu.sync_copy(cl_hbm.at[pl.ds(wid * nclch_w, nclch_w)], clidx)
        pltpu.sync_copy(ones_hbm, ones_v)
        pltpu.sync_copy(src_hbm.at[pl.ds(wid * nch_w, nch_w)], sidx)
        pltpu.sync_copy(dst_hbm.at[pl.ds(wid * nch_w, nch_w)], didx)
        plsc.subcore_barrier()

        def scat(buf, c):
            pltpu.sync_copy(buf, acc_sh.at[didx.at[c]], add=True)
            pltpu.sync_copy(ones_v, deg_sh.at[didx.at[c]], add=True)

        pltpu.async_copy(tab_hbm.at[sidx.at[0]], bufa, sem).wait()

        def pair(p, carry):
            c0 = 2 * p
            d1 = pltpu.async_copy(tab_hbm.at[sidx.at[c0 + 1]], bufb, sem)
            scat(bufa, c0)
            d1.wait()
            cnxt = jnp.minimum(c0 + 2, nch_w - 1)
            d2 = pltpu.async_copy(tab_hbm.at[sidx.at[cnxt]], bufa, sem)
            scat(bufb, c0 + 1)
            d2.wait()
            return carry

        lax.fori_loop(0, npair, pair, 0)
        if tail:
            scat(bufa, nch_w - 1)

        def cstep(q, carry):
            pltpu.sync_copy(ones_v.at[pl.ds(0, _CC)],
                            degc_sh.at[clidx.at[q]], add=True)
            return carry
        lax.fori_loop(0, nclch_w, cstep, 0)

        plsc.subcore_barrier()
        pltpu.sync_copy(acc_sh.at[pl.ds(sid * rpt, rpt)],
                        agg_out.at[cid].at[pl.ds(sid * rpt, rpt)])
        pltpu.sync_copy(deg_sh.at[pl.ds(sid * rpt, rpt)],
                        deg_out.at[cid].at[pl.ds(sid * rpt, rpt)])
        pltpu.sync_copy(degc_sh.at[pl.ds(sid * rpt, rpt)],
                        degc_out.at[cid].at[pl.ds(sid * rpt, rpt)])
    return body


def _g0_body(nch_w, rpt):
    npair = nch_w // 2
    tail = nch_w % 2

    def body(tab_hbm, src_hbm, dst_hbm, z128_hbm, z8_hbm, ones_hbm,
             agg_out, deg_out, sidx, didx, bufa, bufb, ones_v,
             acc_sh, deg_sh, sem):
        cid = lax.axis_index("c")
        sid = lax.axis_index("s")
        wid = cid * _NS + sid
        pltpu.sync_copy(z128_hbm.at[pl.ds(0, rpt)],
                        acc_sh.at[pl.ds(sid * rpt, rpt)])
        pltpu.sync_copy(z8_hbm.at[pl.ds(0, rpt)],
                        deg_sh.at[pl.ds(sid * rpt, rpt)])
        pltpu.sync_copy(ones_hbm, ones_v)
        pltpu.sync_copy(src_hbm.at[pl.ds(wid * nch_w, nch_w)], sidx)
        pltpu.sync_copy(dst_hbm.at[pl.ds(wid * nch_w, nch_w)], didx)
        plsc.subcore_barrier()

        def scat(buf, c):
            pltpu.sync_copy(buf, acc_sh.at[didx.at[c]], add=True)
            pltpu.sync_copy(ones_v, deg_sh.at[didx.at[c]], add=True)

        pltpu.async_copy(tab_hbm.at[sidx.at[0]], bufa, sem).wait()

        def pair(p, carry):
            c0 = 2 * p
            d1 = pltpu.async_copy(tab_hbm.at[sidx.at[c0 + 1]], bufb, sem)
            scat(bufa, c0)
            d1.wait()
            cnxt = jnp.minimum(c0 + 2, nch_w - 1)
            d2 = pltpu.async_copy(tab_hbm.at[sidx.at[cnxt]], bufa, sem)
            scat(bufb, c0 + 1)
            d2.wait()
            return carry

        lax.fori_loop(0, npair, pair, 0)
        if tail:
            scat(bufa, nch_w - 1)
        plsc.subcore_barrier()
        pltpu.sync_copy(acc_sh.at[pl.ds(sid * rpt, rpt)],
                        agg_out.at[cid].at[pl.ds(sid * rpt, rpt)])
        pltpu.sync_copy(deg_sh.at[pl.ds(sid * rpt, rpt)],
                        deg_out.at[cid].at[pl.ds(sid * rpt, rpt)])
    return body


def _inter_body(nch_w, rpt):
    npair = nch_w // 2

    def body(h1_hbm, h0_hbm, cl_hbm, z128_hbm, g1_out, csum_out,
             cidx, ba1, ba0, bb1, bb0, csum_sh, sem1, sem0):
        cid = lax.axis_index("c")
        sid = lax.axis_index("s")
        wid = cid * _NS + sid
        pltpu.sync_copy(z128_hbm.at[pl.ds(0, rpt)],
                        csum_sh.at[pl.ds(sid * rpt, rpt)])
        pltpu.sync_copy(cl_hbm.at[pl.ds(wid * nch_w, nch_w)], cidx)
        plsc.subcore_barrier()

        def issue(c, b1, b0):
            d1 = pltpu.async_copy(h1_hbm.at[cidx.at[c]], b1, sem1)
            d0 = pltpu.async_copy(h0_hbm.at[pl.ds((wid * nch_w + c) * _CC,
                                                  _CC)], b0, sem0)
            return d1, d0

        def drain(c, b1, b0):
            pltpu.sync_copy(b1, g1_out.at[pl.ds((wid * nch_w + c) * _CC,
                                                _CC)])
            pltpu.sync_copy(b0, csum_sh.at[cidx.at[c]], add=True)

        da, db = issue(0, ba1, ba0)
        da.wait()
        db.wait()

        def pair(p, carry):
            c0 = 2 * p
            d1, d0 = issue(c0 + 1, bb1, bb0)
            drain(c0, ba1, ba0)
            d1.wait()
            d0.wait()
            cnxt = jnp.minimum(c0 + 2, nch_w - 1)
            d1, d0 = issue(cnxt, ba1, ba0)
            drain(c0 + 1, bb1, bb0)
            d1.wait()
            d0.wait()
            return carry

        lax.fori_loop(0, npair, pair, 0)
        plsc.subcore_barrier()
        pltpu.sync_copy(csum_sh.at[pl.ds(sid * rpt, rpt)],
                        csum_out.at[cid].at[pl.ds(sid * rpt, rpt)])
    return body


def _h_body(x_ref, p_ref, ws_ref, wn_ref, o_ref):
    p = p_ref[0] + p_ref[1]
    agg = p[:, :_D] * (1.0 / jnp.maximum(p[:, _D:_D + 1], 1.0))
    o_ref[...] = jnp.maximum(
        jnp.dot(x_ref[...], ws_ref[...], preferred_element_type=jnp.float32)
        + jnp.dot(agg, wn_ref[...], preferred_element_type=jnp.float32), 0.0)


def _h0_body(x_ref, p_ref, d_ref, ws_ref, wn_ref, o_ref):
    deg = d_ref[0] + d_ref[1]
    psum = p_ref[0].astype(jnp.float32) + p_ref[1].astype(jnp.float32)
    agg = psum * (1.0 / jnp.maximum(deg[:, :1], 1.0))
    o_ref[...] = jnp.maximum(
        jnp.dot(x_ref[...], ws_ref[...], preferred_element_type=jnp.float32)
        + jnp.dot(agg, wn_ref[...], preferred_element_type=jnp.float32), 0.0)


def _comb0_body(x_ref, h_ref, g_ref, wsi_ref, wni_ref, o_ref):
    h = h_ref[...]
    nz = jnp.maximum(
        jnp.dot(h, wsi_ref[...], preferred_element_type=jnp.float32)
        + jnp.dot(g_ref[...], wni_ref[...], preferred_element_type=jnp.float32),
        0.0)
    o_ref[...] = x_ref[...] + 0.5 * h + 0.5 * nz


def _comb1_body(x_ref, h_ref, c_ref, dc_ref, wsi_ref, wni_ref, o_ref):
    dc = dc_ref[0] + dc_ref[1]
    aggc = (c_ref[0] + c_ref[1]) * (1.0 / jnp.maximum(dc[:, :1], 1.0))
    h = h_ref[...]
    nz = jnp.maximum(
        jnp.dot(h, wsi_ref[...], preferred_element_type=jnp.float32)
        + jnp.dot(aggc, wni_ref[...], preferred_element_type=jnp.float32),
        0.0)
    o_ref[...] = x_ref[...] + 0.5 * h + 0.5 * nz


def _row_spec(bm, d):
    return pl.BlockSpec((bm, d), lambda i: (i, 0))


def _part_spec(bm, d):
    return pl.BlockSpec((2, bm, d), lambda i: (0, i, 0))


def _w_spec():
    return pl.BlockSpec((_D, _D), lambda i: (0, 0))


def _tc_h(xp, parts, ws, wn, n_rows, bm):
    return pl.pallas_call(
        _h_body,
        grid=(n_rows // bm,),
        in_specs=[_row_spec(bm, _D), _part_spec(bm, _DA),
                  _w_spec(), _w_spec()],
        out_specs=_row_spec(bm, _D),
        out_shape=jax.ShapeDtypeStruct((n_rows, _D), jnp.float32),
    )(xp, parts, ws, wn)


def kernel(x0, x1, edge_index0, edge_index1, inter_edge_index,
           W_self0, W_neigh0, W_self1, W_neigh1, W_self_i, W_neigh_i):
    f32 = jnp.float32
    x1p = jnp.pad(x1, ((0, _N1P - _N1), (0, 0)))
    pad1 = _N1 + jnp.arange(_E1P - _E1, dtype=jnp.int32) % (_N1P - _N1)
    padc = _N1 + jnp.arange(_N0P - _N0, dtype=jnp.int32) % (_N1P - _N1)
    src0 = edge_index0[0].reshape(_E0 // _CH0, _CH0)
    dst0 = edge_index0[1].reshape(_E0 // _CH0, _CH0)
    src1 = jnp.pad(edge_index1[0], (0, _E1P - _E1)).reshape(_E1P // _CH1, _CH1)
    dst1 = jnp.concatenate([edge_index1[1], pad1]).reshape(_E1P // _CH1, _CH1)
    cluster = inter_edge_index[1, :_N0] - _N0
    clp = jnp.concatenate([cluster, padc]).reshape(_N0P // _CC, _CC)
    ones16 = jnp.ones((_CH1, _DW), f32)
    ones8 = jnp.ones((_CH0, _DG), f32)
    rpt1 = _N1P // _NS
    rpt0 = _N0 // _NS
    z16 = jnp.zeros((rpt1, _DW), f32)
    z8 = jnp.zeros((rpt0, _DG), f32)
    z128 = jnp.zeros((rpt0, _D), f32)

    nch1 = _E1P // _CH1 // _NW
    nchc = _N0P // _CC // _NW
    nch0 = _E0 // _CH0 // _NW

    bf16 = jnp.bfloat16
    x0h = x0.astype(bf16)
    zb = jnp.zeros((rpt0, _D), bf16)
    agg0, deg0 = pl.kernel(
        _g0_body(nch0, rpt0),
        out_type=(jax.ShapeDtypeStruct((_NC, _N0, _D), bf16),
                  jax.ShapeDtypeStruct((_NC, _N0, _DG), f32)),
        mesh=_mesh,
        scratch_types=[
            pltpu.VMEM((nch0, _CH0), jnp.int32),
            pltpu.VMEM((nch0, _CH0), jnp.int32),
            pltpu.VMEM((_CH0, _D), bf16),
            pltpu.VMEM((_CH0, _D), bf16),
            pltpu.VMEM((_CH0, _DG), f32),
            pltpu.VMEM_SHARED((_N0, _D), bf16),
            pltpu.VMEM_SHARED((_N0, _DG), f32),
            pltpu.SemaphoreType.DMA,
        ],
        name="sc_seg_g0",
        compiler_params=_sc_params,
    )(x0h, src0, dst0, zb, z8, ones8)

    agg1, deg1, degc = pl.kernel(
        _seg_body(nch1, rpt1, nchc),
        out_type=(jax.ShapeDtypeStruct((_NC, _N1P, _D), f32),
                  jax.ShapeDtypeStruct((_NC, _N1P, _DW), f32),
                  jax.ShapeDtypeStruct((_NC, _N1P, _DW), f32)),
        mesh=_mesh,
        scratch_types=[
            pltpu.VMEM((nch1, _CH1), jnp.int32),
            pltpu.VMEM((nch1, _CH1), jnp.int32),
            pltpu.VMEM((nchc, _CC), jnp.int32),
            pltpu.VMEM((_CH1, _D), f32),
            pltpu.VMEM((_CH1, _D), f32),
            pltpu.VMEM((_CH1, _DW), f32),
            pltpu.VMEM_SHARED((_N1P, _D), f32),
            pltpu.VMEM_SHARED((_N1P, _DW), f32),
            pltpu.VMEM_SHARED((_N1P, _DW), f32),
            pltpu.SemaphoreType.DMA,
        ],
        name="sc_seg_g1",
        compiler_params=_sc_params,
    )(x1, src1, dst1, clp, ones16, z128, z16)

    h1p = pl.pallas_call(
        _h0_body,
        grid=(_N1P // _BM1,),
        in_specs=[_row_spec(_BM1, _D), _part_spec(_BM1, _D),
                  _part_spec(_BM1, _DW), _w_spec(), _w_spec()],
        out_specs=_row_spec(_BM1, _D),
        out_shape=jax.ShapeDtypeStruct((_N1P, _D), f32),
    )(x1p, agg1, deg1, W_self1, W_neigh1)
    h0p = pl.pallas_call(
        _h0_body,
        grid=(_N0 // _BM0,),
        in_specs=[_row_spec(_BM0, _D), _part_spec(_BM0, _D),
                  _part_spec(_BM0, _DG), _w_spec(), _w_spec()],
        out_specs=_row_spec(_BM0, _D),
        out_shape=jax.ShapeDtypeStruct((_N0P, _D), f32),
    )(x0, agg0, deg0, W_self0, W_neigh0)

    g1, csum = pl.kernel(
        _inter_body(nchc, rpt1),
        out_type=(jax.ShapeDtypeStruct((_N0P, _D), f32),
                  jax.ShapeDtypeStruct((_NC, _N1P, _D), f32)),
        mesh=_mesh,
        scratch_types=[
            pltpu.VMEM((nchc, _CC), jnp.int32),
            pltpu.VMEM((_CC, _D), f32),
            pltpu.VMEM((_CC, _D), f32),
            pltpu.VMEM((_CC, _D), f32),
            pltpu.VMEM((_CC, _D), f32),
            pltpu.VMEM_SHARED((_N1P, _D), f32),
            pltpu.SemaphoreType.DMA,
            pltpu.SemaphoreType.DMA,
        ],
        name="sc_inter",
        compiler_params=_sc_params,
    )(h1p, h0p, clp, z128)

    out0 = pl.pallas_call(
        _comb0_body,
        grid=(_N0 // _BM0,),
        in_specs=[_row_spec(_BM0, _D), _row_spec(_BM0, _D),
                  _row_spec(_BM0, _D), _w_spec(), _w_spec()],
        out_specs=_row_spec(_BM0, _D),
        out_shape=jax.ShapeDtypeStruct((_N0, _D), f32),
    )(x0, h0p, g1, W_self_i, W_neigh_i)

    out1p = pl.pallas_call(
        _comb1_body,
        grid=(_N1P // _BM1,),
        in_specs=[_row_spec(_BM1, _D), _row_spec(_BM1, _D),
                  _part_spec(_BM1, _D), _part_spec(_BM1, _DW),
                  _w_spec(), _w_spec()],
        out_specs=_row_spec(_BM1, _D),
        out_shape=jax.ShapeDtypeStruct((_N1P, _D), f32),
    )(x1p, h1p, csum, degc, W_self_i, W_neigh_i)

    return (out0, out1p[:_N1])


# speedup vs baseline: 1.2452x; 1.1177x over previous
import jax
import jax.numpy as jnp
from jax import lax
from jax.experimental import pallas as pl
from jax.experimental.pallas import tpu as pltpu
from jax.experimental.pallas import tpu_sc as plsc

_N0, _E0 = 10000, 320000
_N1, _E1 = 2500, 40000
_D = 128
_DA = 144
_NC, _NS = 2, 16
_NW = _NC * _NS
_CH0 = 80
_DG = 8
_CH1 = 128
_CC = 80
_N0P = 10240
_N0A = 10152
_N1P = 2560
_E0P = 323584
_E1P = 40960
_DW = 16
_BM0 = 400
_BM1 = 256

_mesh = plsc.VectorSubcoreMesh(core_axis_name="c", subcore_axis_name="s",
                               num_cores=_NC, num_subcores=_NS)
_sc_params = pltpu.CompilerParams(use_tc_tiling_on_sc=False)


def _seg_body(nch_w, rpt, nclch_w):
    npair = nch_w // 2
    tail = nch_w % 2

    def body(tab_hbm, src_hbm, dst_hbm, cl_hbm, ones_hbm, z128_hbm, z16_hbm,
             agg_out, deg_out, degc_out,
             sidx, didx, clidx, bufa, bufb, ones_v,
             acc_sh, deg_sh, degc_sh, sem):
        cid = lax.axis_index("c")
        sid = lax.axis_index("s")
        wid = cid * _NS + sid
        pltpu.sync_copy(z128_hbm.at[pl.ds(0, rpt)],
                        acc_sh.at[pl.ds(sid * rpt, rpt)])
        pltpu.sync_copy(z16_hbm.at[pl.ds(0, rpt)],
                        deg_sh.at[pl.ds(sid * rpt, rpt)])
        pltpu.sync_copy(z16_hbm.at[pl.ds(0, rpt)],
                        degc_sh.at[pl.ds(sid * rpt, rpt)])
        pltpu.sync_copy(cl_hbm.at[pl.ds(wid * nclch_w, nclch_w)], clidx)
        pltpu.sync_copy(ones_hbm, ones_v)
        pltpu.sync_copy(src_hbm.at[pl.ds(wid * nch_w, nch_w)], sidx)
        pltpu.sync_copy(dst_hbm.at[pl.ds(wid * nch_w, nch_w)], didx)
        plsc.subcore_barrier()

        def scat(buf, c):
            pltpu.sync_copy(buf, acc_sh.at[didx.at[c]], add=True)
            pltpu.sync_copy(ones_v, deg_sh.at[didx.at[c]], add=True)

        pltpu.async_copy(tab_hbm.at[sidx.at[0]], bufa, sem).wait()

        def pair(p, carry):
            c0 = 2 * p
            d1 = pltpu.async_copy(tab_hbm.at[sidx.at[c0 + 1]], bufb, sem)
            scat(bufa, c0)
            d1.wait()
            cnxt = jnp.minimum(c0 + 2, nch_w - 1)
            d2 = pltpu.async_copy(tab_hbm.at[sidx.at[cnxt]], bufa, sem)
            scat(bufb, c0 + 1)
            d2.wait()
            return carry

        lax.fori_loop(0, npair, pair, 0)
        if tail:
            scat(bufa, nch_w - 1)

        def cstep(q, carry):
            pltpu.sync_copy(ones_v.at[pl.ds(0, _CC)],
                            degc_sh.at[clidx.at[q]], add=True)
            return carry
        lax.fori_loop(0, nclch_w, cstep, 0)

        plsc.subcore_barrier()
        pltpu.sync_copy(acc_sh.at[pl.ds(sid * rpt, rpt)],
                        agg_out.at[cid].at[pl.ds(sid * rpt, rpt)])
        pltpu.sync_copy(deg_sh.at[pl.ds(sid * rpt, rpt)],
                        deg_out.at[cid].at[pl.ds(sid * rpt, rpt)])
        pltpu.sync_copy(degc_sh.at[pl.ds(sid * rpt, rpt)],
                        degc_out.at[cid].at[pl.ds(sid * rpt, rpt)])
    return body


def _g0_body(nch_w, rpt):
    npair = nch_w // 2
    tail = nch_w % 2

    def body(tab_hbm, src_hbm, dst_hbm, z128_hbm, z8_hbm, ones_hbm,
             agg_out, deg_out, sidx, didx, bufa, bufb, ones_v,
             acc_sh, deg_sh, sem):
        cid = lax.axis_index("c")
        sid = lax.axis_index("s")
        wid = cid * _NS + sid
        pltpu.sync_copy(z128_hbm.at[pl.ds(0, rpt)],
                        acc_sh.at[pl.ds(sid * rpt, rpt)])
        pltpu.sync_copy(z8_hbm.at[pl.ds(0, rpt)],
                        deg_sh.at[pl.ds(sid * rpt, rpt)])
        pltpu.sync_copy(ones_hbm, ones_v)
        pltpu.sync_copy(src_hbm.at[pl.ds(wid * nch_w, nch_w)], sidx)
        pltpu.sync_copy(dst_hbm.at[pl.ds(wid * nch_w, nch_w)], didx)
        plsc.subcore_barrier()

        def scat(buf, c):
            pltpu.sync_copy(buf, acc_sh.at[didx.at[c]], add=True)
            pltpu.sync_copy(ones_v, deg_sh.at[didx.at[c]], add=True)

        pltpu.async_copy(tab_hbm.at[sidx.at[0]], bufa, sem).wait()

        def pair(p, carry):
            c0 = 2 * p
            d1 = pltpu.async_copy(tab_hbm.at[sidx.at[c0 + 1]], bufb, sem)
            scat(bufa, c0)
            d1.wait()
            cnxt = jnp.minimum(c0 + 2, nch_w - 1)
            d2 = pltpu.async_copy(tab_hbm.at[sidx.at[cnxt]], bufa, sem)
            scat(bufb, c0 + 1)
            d2.wait()
            return carry

        lax.fori_loop(0, npair, pair, 0)
        if tail:
            scat(bufa, nch_w - 1)
        plsc.subcore_barrier()
        pltpu.sync_copy(acc_sh.at[pl.ds(sid * rpt, rpt)],
                        agg_out.at[cid].at[pl.ds(sid * rpt, rpt)])
        pltpu.sync_copy(deg_sh.at[pl.ds(sid * rpt, rpt)],
                        deg_out.at[cid].at[pl.ds(sid * rpt, rpt)])
    return body


def _inter_body(nch_w, rpt):
    npair = nch_w // 2

    def body(h1_hbm, h0_hbm, cl_hbm, z128_hbm, g1_out, csum_out,
             cidx, ba1, ba0, bb1, bb0, csum_sh, sem1, sem0):
        cid = lax.axis_index("c")
        sid = lax.axis_index("s")
        wid = cid * _NS + sid
        pltpu.sync_copy(z128_hbm.at[pl.ds(0, rpt)],
                        csum_sh.at[pl.ds(sid * rpt, rpt)])
        pltpu.sync_copy(cl_hbm.at[pl.ds(wid * nch_w, nch_w)], cidx)
        plsc.subcore_barrier()

        def issue(c, b1, b0):
            d1 = pltpu.async_copy(h1_hbm.at[cidx.at[c]], b1, sem1)
            d0 = pltpu.async_copy(h0_hbm.at[pl.ds((wid * nch_w + c) * _CC,
                                                  _CC)], b0, sem0)
            return d1, d0

        def drain(c, b1, b0):
            pltpu.sync_copy(b1, g1_out.at[pl.ds((wid * nch_w + c) * _CC,
                                                _CC)])
            pltpu.sync_copy(b0, csum_sh.at[cidx.at[c]], add=True)

        da, db = issue(0, ba1, ba0)
        da.wait()
        db.wait()

        def pair(p, carry):
            c0 = 2 * p
            d1, d0 = issue(c0 + 1, bb1, bb0)
            drain(c0, ba1, ba0)
            d1.wait()
            d0.wait()
            cnxt = jnp.minimum(c0 + 2, nch_w - 1)
            d1, d0 = issue(cnxt, ba1, ba0)
            drain(c0 + 1, bb1, bb0)
            d1.wait()
            d0.wait()
            return carry

        lax.fori_loop(0, npair, pair, 0)
        plsc.subcore_barrier()
        pltpu.sync_copy(csum_sh.at[pl.ds(sid * rpt, rpt)],
                        csum_out.at[cid].at[pl.ds(sid * rpt, rpt)])
    return body


def _h_body(x_ref, p_ref, ws_ref, wn_ref, o_ref):
    p = p_ref[0] + p_ref[1]
    agg = p[:, :_D] * (1.0 / jnp.maximum(p[:, _D:_D + 1], 1.0))
    o_ref[...] = jnp.maximum(
        jnp.dot(x_ref[...], ws_ref[...], preferred_element_type=jnp.float32)
        + jnp.dot(agg, wn_ref[...], preferred_element_type=jnp.float32), 0.0)


def _h0_body(x_ref, p_ref, d_ref, ws_ref, wn_ref, o_ref):
    deg = d_ref[0] + d_ref[1]
    psum = p_ref[0].astype(jnp.float32) + p_ref[1].astype(jnp.float32)
    agg = psum * (1.0 / jnp.maximum(deg[:, :1], 1.0))
    o_ref[...] = jnp.maximum(
        jnp.dot(x_ref[...], ws_ref[...], preferred_element_type=jnp.float32)
        + jnp.dot(agg, wn_ref[...], preferred_element_type=jnp.float32), 0.0)


def _comb0_body(x_ref, h_ref, g_ref, wsi_ref, wni_ref, o_ref):
    h = h_ref[...]
    nz = jnp.maximum(
        jnp.dot(h, wsi_ref[...], preferred_element_type=jnp.float32)
        + jnp.dot(g_ref[...], wni_ref[...], preferred_element_type=jnp.float32),
        0.0)
    o_ref[...] = x_ref[...] + 0.5 * h + 0.5 * nz


def _comb1_body(x_ref, h_ref, c_ref, dc_ref, wsi_ref, wni_ref, o_ref):
    dc = dc_ref[0] + dc_ref[1]
    aggc = (c_ref[0] + c_ref[1]) * (1.0 / jnp.maximum(dc[:, :1], 1.0))
    h = h_ref[...]
    nz = jnp.maximum(
        jnp.dot(h, wsi_ref[...], preferred_element_type=jnp.float32)
        + jnp.dot(aggc, wni_ref[...], preferred_element_type=jnp.float32),
        0.0)
    o_ref[...] = x_ref[...] + 0.5 * h + 0.5 * nz


def _row_spec(bm, d):
    return pl.BlockSpec((bm, d), lambda i: (i, 0))


def _part_spec(bm, d):
    return pl.BlockSpec((2, bm, d), lambda i: (0, i, 0))


def _w_spec():
    return pl.BlockSpec((_D, _D), lambda i: (0, 0))


def _tc_h(xp, parts, ws, wn, n_rows, bm):
    return pl.pallas_call(
        _h_body,
        grid=(n_rows // bm,),
        in_specs=[_row_spec(bm, _D), _part_spec(bm, _DA),
                  _w_spec(), _w_spec()],
        out_specs=_row_spec(bm, _D),
        out_shape=jax.ShapeDtypeStruct((n_rows, _D), jnp.float32),
    )(xp, parts, ws, wn)


def kernel(x0, x1, edge_index0, edge_index1, inter_edge_index,
           W_self0, W_neigh0, W_self1, W_neigh1, W_self_i, W_neigh_i):
    f32 = jnp.float32
    x1p = jnp.pad(x1, ((0, _N1P - _N1), (0, 0)))
    pad1 = _N1 + jnp.arange(_E1P - _E1, dtype=jnp.int32) % (_N1P - _N1)
    padc = _N1 + jnp.arange(_N0P - _N0, dtype=jnp.int32) % (_N1P - _N1)
    src0 = edge_index0[0].reshape(_E0 // _CH0, _CH0)
    dst0 = edge_index0[1].reshape(_E0 // _CH0, _CH0)
    src1 = jnp.pad(edge_index1[0], (0, _E1P - _E1)).reshape(_E1P // _CH1, _CH1)
    dst1 = jnp.concatenate([edge_index1[1], pad1]).reshape(_E1P // _CH1, _CH1)
    cluster = inter_edge_index[1, :_N0] - _N0
    clp = jnp.concatenate([cluster, padc]).reshape(_N0P // _CC, _CC)
    ones16 = jnp.ones((_CH1, _DW), f32)
    ones8 = jnp.ones((_CH0, _DG), f32)
    rpt1 = _N1P // _NS
    rpt0 = _N0 // _NS
    z16 = jnp.zeros((rpt1, _DW), f32)
    z8 = jnp.zeros((rpt0, _DG), f32)
    z128 = jnp.zeros((rpt0, _D), f32)

    nch1 = _E1P // _CH1 // _NW
    nchc = _N0P // _CC // _NW
    nch0 = _E0 // _CH0 // _NW

    bf16 = jnp.bfloat16
    x0h = x0.astype(bf16)
    zb = jnp.zeros((rpt0, _D), bf16)
    agg0, deg0 = pl.kernel(
        _g0_body(nch0, rpt0),
        out_type=(jax.ShapeDtypeStruct((_NC, _N0, _D), bf16),
                  jax.ShapeDtypeStruct((_NC, _N0, _DG), f32)),
        mesh=_mesh,
        scratch_types=[
            pltpu.VMEM((nch0, _CH0), jnp.int32),
            pltpu.VMEM((nch0, _CH0), jnp.int32),
            pltpu.VMEM((_CH0, _D), bf16),
            pltpu.VMEM((_CH0, _D), bf16),
            pltpu.VMEM((_CH0, _DG), f32),
            pltpu.VMEM_SHARED((_N0, _D), bf16),
            pltpu.VMEM_SHARED((_N0, _DG), f32),
            pltpu.SemaphoreType.DMA,
        ],
        name="sc_seg_g0",
        compiler_params=_sc_params,
    )(x0h, src0, dst0, zb, z8, ones8)

    x1h = x1.astype(bf16)
    agg1, deg1, degc = pl.kernel(
        _seg_body(nch1, rpt1, nchc),
        out_type=(jax.ShapeDtypeStruct((_NC, _N1P, _D), bf16),
                  jax.ShapeDtypeStruct((_NC, _N1P, _DW), f32),
                  jax.ShapeDtypeStruct((_NC, _N1P, _DW), f32)),
        mesh=_mesh,
        scratch_types=[
            pltpu.VMEM((nch1, _CH1), jnp.int32),
            pltpu.VMEM((nch1, _CH1), jnp.int32),
            pltpu.VMEM((nchc, _CC), jnp.int32),
            pltpu.VMEM((_CH1, _D), bf16),
            pltpu.VMEM((_CH1, _D), bf16),
            pltpu.VMEM((_CH1, _DW), f32),
            pltpu.VMEM_SHARED((_N1P, _D), bf16),
            pltpu.VMEM_SHARED((_N1P, _DW), f32),
            pltpu.VMEM_SHARED((_N1P, _DW), f32),
            pltpu.SemaphoreType.DMA,
        ],
        name="sc_seg_g1",
        compiler_params=_sc_params,
    )(x1h, src1, dst1, clp, ones16, zb, z16)

    h1p = pl.pallas_call(
        _h0_body,
        grid=(_N1P // _BM1,),
        in_specs=[_row_spec(_BM1, _D), _part_spec(_BM1, _D),
                  _part_spec(_BM1, _DW), _w_spec(), _w_spec()],
        out_specs=_row_spec(_BM1, _D),
        out_shape=jax.ShapeDtypeStruct((_N1P, _D), f32),
    )(x1p, agg1, deg1, W_self1, W_neigh1)
    h0p = pl.pallas_call(
        _h0_body,
        grid=(_N0 // _BM0,),
        in_specs=[_row_spec(_BM0, _D), _part_spec(_BM0, _D),
                  _part_spec(_BM0, _DG), _w_spec(), _w_spec()],
        out_specs=_row_spec(_BM0, _D),
        out_shape=jax.ShapeDtypeStruct((_N0P, _D), f32),
    )(x0, agg0, deg0, W_self0, W_neigh0)

    g1, csum = pl.kernel(
        _inter_body(nchc, rpt1),
        out_type=(jax.ShapeDtypeStruct((_N0P, _D), f32),
                  jax.ShapeDtypeStruct((_NC, _N1P, _D), f32)),
        mesh=_mesh,
        scratch_types=[
            pltpu.VMEM((nchc, _CC), jnp.int32),
            pltpu.VMEM((_CC, _D), f32),
            pltpu.VMEM((_CC, _D), f32),
            pltpu.VMEM((_CC, _D), f32),
            pltpu.VMEM((_CC, _D), f32),
            pltpu.VMEM_SHARED((_N1P, _D), f32),
            pltpu.SemaphoreType.DMA,
            pltpu.SemaphoreType.DMA,
        ],
        name="sc_inter",
        compiler_params=_sc_params,
    )(h1p, h0p, clp, z128)

    out0 = pl.pallas_call(
        _comb0_body,
        grid=(_N0 // _BM0,),
        in_specs=[_row_spec(_BM0, _D), _row_spec(_BM0, _D),
                  _row_spec(_BM0, _D), _w_spec(), _w_spec()],
        out_specs=_row_spec(_BM0, _D),
        out_shape=jax.ShapeDtypeStruct((_N0, _D), f32),
    )(x0, h0p, g1, W_self_i, W_neigh_i)

    out1p = pl.pallas_call(
        _comb1_body,
        grid=(_N1P // _BM1,),
        in_specs=[_row_spec(_BM1, _D), _row_spec(_BM1, _D),
                  _part_spec(_BM1, _D), _part_spec(_BM1, _DW),
                  _w_spec(), _w_spec()],
        out_specs=_row_spec(_BM1, _D),
        out_shape=jax.ShapeDtypeStruct((_N1P, _D), f32),
    )(x1p, h1p, csum, degc, W_self_i, W_neigh_i)

    return (out0, out1p[:_N1])
